# baseline (speedup 1.0000x reference)
import functools

import numpy as np
import jax
import jax.numpy as jnp
from jax import lax
from jax.experimental import pallas as pl
from jax.experimental.pallas import tpu as pltpu
from jax.experimental.pallas import tpu_sc as plsc

_BATCH = 16384
_NFIELD = 26
_DIM = 16
_ROWS = 1040000
_FIELD_SIZE = 40000
_NC = 2
_NS = 16
_NW = _NC * _NS
_BPW = _BATCH // _NW
_RT = _ROWS // 128
_LANES = 16
_NG = _BPW // _LANES
_NBLK = -(-_RT // _NW)


@functools.partial(
    pl.kernel,
    mesh=plsc.VectorSubcoreMesh(core_axis_name="c", subcore_axis_name="s"),
    compiler_params=pltpu.CompilerParams(use_tc_tiling_on_sc=False,
                                         needs_layout_passes=False),
    out_type=jax.ShapeDtypeStruct((_ROWS * _DIM,), jnp.float32),
    scratch_types=[
        pltpu.VMEM((2048,), jnp.float32),
        pltpu.VMEM((2048,), jnp.float32),
        pltpu.VMEM((2048,), jnp.float32),
        pltpu.VMEM((2048,), jnp.float32),
        pltpu.SemaphoreType.DMA,
        pltpu.SemaphoreType.DMA,
        pltpu.SemaphoreType.DMA,
        pltpu.SemaphoreType.DMA,
    ],
)
def _transpose_table(tbl_hbm, out_hbm, vt0, vt1, ob0, ob1,
                     isem0, isem1, osem0, osem1):
    wid = lax.axis_index("s") * _NC + lax.axis_index("c")
    vts = (vt0, vt1)
    obs = (ob0, ob1)
    isems = (isem0, isem1)
    osems = (osem0, osem1)

    i16 = lax.iota(jnp.int32, 16)
    abase = (i16 & 7) * 128 + (i16 >> 3) * 1024

    def blk(k):
        return jnp.minimum(wid + k * _NW, _RT - 1)

    def issue_in(k, j):
        rt = blk(k)
        pltpu.async_copy(tbl_hbm.at[pl.ds(rt * 1024, 1024)],
                         vts[j].at[pl.ds(0, 1024)], isems[j])
        pltpu.async_copy(tbl_hbm.at[pl.ds((_RT + rt) * 1024, 1024)],
                         vts[j].at[pl.ds(1024, 1024)], isems[j])

    def drain_in(j):
        for h in range(2):
            pltpu.make_async_copy(tbl_hbm.at[pl.ds(0, 1024)],
                                  vts[j].at[pl.ds(h * 1024, 1024)],
                                  isems[j]).wait()

    def issue_out(k, j):
        rt = blk(k)
        pltpu.async_copy(obs[j].at[pl.ds(0, 2048)],
                         out_hbm.at[pl.ds(rt * 2048, 2048)], osems[j])

    def drain_out(j):
        pltpu.make_async_copy(obs[j].at[pl.ds(0, 2048)],
                              out_hbm.at[pl.ds(0, 2048)], osems[j]).wait()

    def shuffle(j):
        def body(it, carry):
            rl0 = it * 8
            for u in range(8):
                row = plsc.load_gather(vts[j], [abase + (rl0 + u)])
                obs[j][pl.ds((rl0 + u) * _DIM, _LANES)] = row
            return carry
        lax.fori_loop(0, 16, body, 0)

    for j in range(2):
        issue_in(j, j)

    def loop(k2, carry):
        for j in range(2):
            k = k2 * 2 + j
            drain_in(j)

            @pl.when(k2 > 0)
            def _():
                drain_out(j)

            shuffle(j)
            issue_out(k, j)
            issue_in(k + 2, j)
        return carry
    lax.fori_loop(0, _NBLK // 2, loop, 0)

    for j in range(2):
        drain_in(j)
        drain_out(j)


@functools.partial(
    pl.kernel,
    mesh=plsc.VectorSubcoreMesh(core_axis_name="c", subcore_axis_name="s"),
    compiler_params=pltpu.CompilerParams(use_tc_tiling_on_sc=False,
                                         needs_layout_passes=False),
    out_type=jax.ShapeDtypeStruct((_NFIELD * 2 * 131072,), jnp.float32),
    scratch_types=[
        pltpu.VMEM((_BPW,), jnp.int32),
        pltpu.VMEM((_BPW,), jnp.int32),
        pltpu.VMEM((_BPW, _DIM), jnp.float32),
        pltpu.VMEM((_BPW, _DIM), jnp.float32),
        pltpu.VMEM((8192,), jnp.float32),
        pltpu.VMEM((8192,), jnp.float32),
        pltpu.SemaphoreType.DMA,
        pltpu.SemaphoreType.DMA,
        pltpu.SemaphoreType.DMA,
        pltpu.SemaphoreType.DMA,
    ],
)
def _row_gather(xt_hbm, tbl2_hbm, out_hbm,
                idx0, idx1, land0, land1, tbuf0, tbuf1,
                gsem0, gsem1, osem0, osem1):
    wid = lax.axis_index("s") * _NC + lax.axis_index("c")
    b0 = wid * _BPW
    bt0 = wid * (_BPW // 128)

    i16 = lax.iota(jnp.int32, 16)

    idxs = (idx0, idx1)
    lands = (land0, land1)
    tbufs = (tbuf0, tbuf1)
    gsems = (gsem0, gsem1)
    osems = (osem0, osem1)

    def build_idx(f, p):
        pltpu.sync_copy(xt_hbm.at[pl.ds(f * _BATCH + b0, _BPW)], idxs[p])
        foff = f * _FIELD_SIZE

        def wb(g, carry):
            s = g * _LANES
            idxs[p][pl.ds(s, _LANES)] = idxs[p][pl.ds(s, _LANES)] + foff
            return carry
        lax.fori_loop(0, _NG, wb, 0)

    ecols = [i16 * 0 + e for e in range(_DIM)]

    def transpose(p):
        def body(g, carry):
            bt = g >> 3
            blg = g & 7
            bstart = bt * 128 + blg * _LANES
            rows = i16 + bstart
            dbase = bt * 1024 + blg * _LANES
            for eg in range(2):
                for es in range(8):
                    row = plsc.load_gather(lands[p], [rows, ecols[eg * 8 + es]])
                    tbufs[p][pl.ds(dbase + eg * 4096 + es * 128, _LANES)] = row
            return carry
        lax.fori_loop(0, _NG, body, 0)

    build_idx(0, 0)
    g_prev = pltpu.async_copy(tbl2_hbm.at[idx0], land0, gsem0)
    o_prev = [None, None]
    for f in range(_NFIELD):
        p = f % 2
        q = (f + 1) % 2
        if f + 1 < _NFIELD:
            build_idx(f + 1, q)
            g_next = pltpu.async_copy(tbl2_hbm.at[idxs[q]], lands[q], gsems[q])
        g_prev.wait()
        if o_prev[p] is not None:
            o_prev[p][0].wait()
            o_prev[p][1].wait()
        transpose(p)
        o_prev[p] = (
            pltpu.async_copy(tbufs[p].at[pl.ds(0, 4096)],
                             out_hbm.at[pl.ds(f * 262144 + bt0 * 1024, 4096)],
                             osems[p]),
            pltpu.async_copy(tbufs[p].at[pl.ds(4096, 4096)],
                             out_hbm.at[pl.ds(f * 262144 + 131072 + bt0 * 1024,
                                              4096)],
                             osems[p]),
        )
        if f + 1 < _NFIELD:
            g_prev = g_next
    for p in range(2):
        if o_prev[p] is not None:
            o_prev[p][0].wait()
            o_prev[p][1].wait()


def kernel(x, table):
    tbl = table.T.reshape(2, 8, _RT, 128).transpose(0, 2, 1, 3).reshape(-1)
    xt = x.T.reshape(-1)
    tbl_rm = _transpose_table(tbl).reshape(_ROWS, _DIM)
    out1 = _row_gather(xt, tbl_rm)
    out5 = out1.reshape(_NFIELD, 2, 128, 8, 128)
    return out5.transpose(2, 4, 0, 1, 3).reshape(_BATCH, _NFIELD, _DIM)


# speedup vs baseline: 1.7438x; 1.7438x over previous
import functools

import numpy as np
import jax
import jax.numpy as jnp
from jax import lax
from jax.experimental import pallas as pl
from jax.experimental.pallas import tpu as pltpu
from jax.experimental.pallas import tpu_sc as plsc

_BATCH = 16384
_NFIELD = 26
_DIM = 16
_ROWS = 1040000
_FIELD_SIZE = 40000
_NC = 2
_NS = 16
_NW = _NC * _NS
_BPW = _BATCH // _NW
_RT = _ROWS // 128
_LANES = 16
_NG = _BPW // _LANES
_NBLK = -(-_RT // _NW)


@functools.partial(
    pl.kernel,
    mesh=plsc.VectorSubcoreMesh(core_axis_name="c", subcore_axis_name="s"),
    compiler_params=pltpu.CompilerParams(use_tc_tiling_on_sc=False,
                                         needs_layout_passes=False),
    out_type=jax.ShapeDtypeStruct((_ROWS * 24,), jnp.float32),
    scratch_types=[
        pltpu.VMEM((2176,), jnp.float32),
        pltpu.VMEM((2176,), jnp.float32),
        pltpu.VMEM((3072,), jnp.float32),
        pltpu.VMEM((3072,), jnp.float32),
        pltpu.SemaphoreType.DMA,
        pltpu.SemaphoreType.DMA,
        pltpu.SemaphoreType.DMA,
        pltpu.SemaphoreType.DMA,
    ],
)
def _transpose_table(tbl_hbm, out_hbm, vt0, vt1, ob0, ob1,
                     isem0, isem1, osem0, osem1):
    wid = lax.axis_index("s") * _NC + lax.axis_index("c")
    vts = (vt0, vt1)
    obs = (ob0, ob1)
    isems = (isem0, isem1)
    osems = (osem0, osem1)

    i16 = lax.iota(jnp.int32, 16)
    abase = (i16 & 7) * 136 + (i16 >> 3) * 1088

    def blk(k):
        return jnp.minimum(wid + k * _NW, _RT - 1)

    def issue_in(k, j):
        rt = blk(k)
        for es in range(8):
            pltpu.async_copy(tbl_hbm.at[pl.ds(rt * 1024 + es * 128, 128)],
                             vts[j].at[pl.ds(es * 136, 128)], isems[j])
            pltpu.async_copy(
                tbl_hbm.at[pl.ds((_RT + rt) * 1024 + es * 128, 128)],
                vts[j].at[pl.ds(1088 + es * 136, 128)], isems[j])

    def drain_in(j):
        pltpu.make_async_copy(tbl_hbm.at[pl.ds(0, 2048)],
                              vts[j].at[pl.ds(0, 2048)], isems[j]).wait()

    def issue_out(k, j):
        rt = blk(k)
        pltpu.async_copy(obs[j].at[pl.ds(0, 3072)],
                         out_hbm.at[pl.ds(rt * 3072, 3072)], osems[j])

    def drain_out(j):
        pltpu.make_async_copy(obs[j].at[pl.ds(0, 3072)],
                              out_hbm.at[pl.ds(0, 3072)], osems[j]).wait()

    def shuffle(j):
        def body(it, carry):
            rl0 = it * 8
            for u in range(8):
                row = plsc.load_gather(vts[j], [abase + (rl0 + u)])
                obs[j][pl.ds((rl0 + u) * 24, _LANES)] = row
            return carry
        lax.fori_loop(0, 16, body, 0)

    for j in range(2):
        issue_in(j, j)

    def loop(k2, carry):
        for j in range(2):
            k = k2 * 2 + j
            drain_in(j)

            @pl.when(k2 > 0)
            def _():
                drain_out(j)

            shuffle(j)
            issue_out(k, j)
            issue_in(k + 2, j)
        return carry
    lax.fori_loop(0, _NBLK // 2, loop, 0)

    for j in range(2):
        drain_in(j)
        drain_out(j)


@functools.partial(
    pl.kernel,
    mesh=plsc.VectorSubcoreMesh(core_axis_name="c", subcore_axis_name="s"),
    compiler_params=pltpu.CompilerParams(use_tc_tiling_on_sc=False,
                                         needs_layout_passes=False),
    out_type=jax.ShapeDtypeStruct((_NFIELD * 2 * 131072,), jnp.float32),
    scratch_types=[
        pltpu.VMEM((_BPW,), jnp.int32),
        pltpu.VMEM((_BPW,), jnp.int32),
        pltpu.VMEM((_BPW, 24), jnp.float32),
        pltpu.VMEM((_BPW, 24), jnp.float32),
        pltpu.VMEM((8192,), jnp.float32),
        pltpu.VMEM((8192,), jnp.float32),
        pltpu.SemaphoreType.DMA,
        pltpu.SemaphoreType.DMA,
        pltpu.SemaphoreType.DMA,
        pltpu.SemaphoreType.DMA,
    ],
)
def _row_gather(xt_hbm, tbl2_hbm, out_hbm,
                idx0, idx1, land0, land1, tbuf0, tbuf1,
                gsem0, gsem1, osem0, osem1):
    wid = lax.axis_index("s") * _NC + lax.axis_index("c")
    b0 = wid * _BPW
    bt0 = wid * (_BPW // 128)

    i16 = lax.iota(jnp.int32, 16)

    idxs = (idx0, idx1)
    lands = (land0, land1)
    tbufs = (tbuf0, tbuf1)
    gsems = (gsem0, gsem1)
    osems = (osem0, osem1)

    def build_idx(f, p):
        pltpu.sync_copy(xt_hbm.at[pl.ds(f * _BATCH + b0, _BPW)], idxs[p])
        foff = f * _FIELD_SIZE

        def wb(g, carry):
            s = g * _LANES
            idxs[p][pl.ds(s, _LANES)] = idxs[p][pl.ds(s, _LANES)] + foff
            return carry
        lax.fori_loop(0, _NG, wb, 0)

    ecols = [i16 * 0 + e for e in range(_DIM)]

    def transpose(p):
        def body(g, carry):
            bt = g >> 3
            blg = g & 7
            bstart = bt * 128 + blg * _LANES
            rows = i16 + bstart
            dbase = bt * 1024 + blg * _LANES
            for eg in range(2):
                for es in range(8):
                    row = plsc.load_gather(lands[p], [rows, ecols[eg * 8 + es]])
                    tbufs[p][pl.ds(dbase + eg * 4096 + es * 128, _LANES)] = row
            return carry
        lax.fori_loop(0, _NG, body, 0)

    build_idx(0, 0)
    g_prev = pltpu.async_copy(tbl2_hbm.at[idx0], land0, gsem0)
    o_prev = [None, None]
    for f in range(_NFIELD):
        p = f % 2
        q = (f + 1) % 2
        if f + 1 < _NFIELD:
            build_idx(f + 1, q)
            g_next = pltpu.async_copy(tbl2_hbm.at[idxs[q]], lands[q], gsems[q])
        g_prev.wait()
        if o_prev[p] is not None:
            o_prev[p][0].wait()
            o_prev[p][1].wait()
        transpose(p)
        o_prev[p] = (
            pltpu.async_copy(tbufs[p].at[pl.ds(0, 4096)],
                             out_hbm.at[pl.ds(f * 262144 + bt0 * 1024, 4096)],
                             osems[p]),
            pltpu.async_copy(tbufs[p].at[pl.ds(4096, 4096)],
                             out_hbm.at[pl.ds(f * 262144 + 131072 + bt0 * 1024,
                                              4096)],
                             osems[p]),
        )
        if f + 1 < _NFIELD:
            g_prev = g_next
    for p in range(2):
        if o_prev[p] is not None:
            o_prev[p][0].wait()
            o_prev[p][1].wait()


def kernel(x, table):
    tbl = table.T.reshape(2, 8, _RT, 128).transpose(0, 2, 1, 3).reshape(-1)
    xt = x.T.reshape(-1)
    tbl_rm = _transpose_table(tbl).reshape(_ROWS, 24)
    out1 = _row_gather(xt, tbl_rm)
    out5 = out1.reshape(_NFIELD, 2, 128, 8, 128)
    return out5.transpose(2, 4, 0, 1, 3).reshape(_BATCH, _NFIELD, _DIM)


# speedup vs baseline: 1.9439x; 1.1147x over previous
import functools

import numpy as np
import jax
import jax.numpy as jnp
from jax import lax
from jax.experimental import pallas as pl
from jax.experimental.pallas import tpu as pltpu
from jax.experimental.pallas import tpu_sc as plsc

_BATCH = 16384
_NFIELD = 26
_DIM = 16
_ROWS = 1040000
_FIELD_SIZE = 40000
_NC = 2
_NS = 16
_NW = _NC * _NS
_BPW = _BATCH // _NW
_RT = _ROWS // 128
_LANES = 16
_NG = _BPW // _LANES

_EBASE = np.array([(e // 8) * _RT * 1024 + (e % 8) * 128 for e in range(_DIM)],
                  dtype=np.int32)


@functools.partial(
    pl.kernel,
    mesh=plsc.VectorSubcoreMesh(core_axis_name="c", subcore_axis_name="s"),
    compiler_params=pltpu.CompilerParams(use_tc_tiling_on_sc=False),
    out_type=jax.ShapeDtypeStruct((_NFIELD * 2 * 131072,), jnp.float32),
    scratch_types=[
        pltpu.VMEM((_BPW,), jnp.int32),
        pltpu.VMEM((_BPW,), jnp.int32),
        pltpu.VMEM((8192,), jnp.int32),
        pltpu.VMEM((8192,), jnp.int32),
        pltpu.VMEM((8192,), jnp.float32),
        pltpu.VMEM((8192,), jnp.float32),
        pltpu.SemaphoreType.DMA,
        pltpu.SemaphoreType.DMA,
        pltpu.SemaphoreType.DMA,
        pltpu.SemaphoreType.DMA,
    ],
)
def _embed_gather(xt_hbm, tbl_hbm, out_hbm,
                  xbuf, wbase, idx0, idx1, land0, land1,
                  gsem0, gsem1, osem0, osem1):
    wid = lax.axis_index("s") * _NC + lax.axis_index("c")
    b0 = wid * _BPW
    bt0 = wid * (_BPW // 128)

    def build_indices(f, idx_v):
        pltpu.sync_copy(xt_hbm.at[pl.ds(f * _BATCH + b0, _BPW)], xbuf)
        foff = f * _FIELD_SIZE

        def wb(g, carry):
            s = g * _LANES
            r = xbuf[pl.ds(s, _LANES)] + foff
            wbase[pl.ds(s, _LANES)] = ((r >> 7) << 10) + (r & 127)
            return carry
        lax.fori_loop(0, _NG, wb, 0)

        def grp(g, carry):
            base = wbase[pl.ds(g * _LANES, _LANES)]
            bt = g // 8
            blg = g % 8
            for eg in range(2):
                for es in range(8):
                    dst = (eg * 4 + bt) * 1024 + es * 128 + blg * _LANES
                    idx_v[pl.ds(dst, _LANES)] = base + int(_EBASE[eg * 8 + es])
            return carry
        lax.fori_loop(0, _NG, grp, 0)

    idxs = (idx0, idx1)
    lands = (land0, land1)
    gsems = (gsem0, gsem1)
    osems = (osem0, osem1)

    build_indices(0, idx0)
    g_prev = pltpu.async_copy(tbl_hbm.at[idx0], land0, gsem0)
    o_prev = [None, None]
    for f in range(_NFIELD):
        p = f % 2
        q = (f + 1) % 2
        if f + 1 < _NFIELD:
            if o_prev[q] is not None:
                o_prev[q][0].wait()
                o_prev[q][1].wait()
            build_indices(f + 1, idxs[q])
            g_next = pltpu.async_copy(tbl_hbm.at[idxs[q]], lands[q], gsems[q])
        g_prev.wait()
        o_prev[p] = (
            pltpu.async_copy(lands[p].at[pl.ds(0, 4096)],
                             out_hbm.at[pl.ds(f * 262144 + bt0 * 1024, 4096)],
                             osems[p]),
            pltpu.async_copy(lands[p].at[pl.ds(4096, 4096)],
                             out_hbm.at[pl.ds(f * 262144 + 131072 + bt0 * 1024,
                                              4096)],
                             osems[p]),
        )
        if f + 1 < _NFIELD:
            g_prev = g_next
    for p in range(2):
        if o_prev[p] is not None:
            o_prev[p][0].wait()
            o_prev[p][1].wait()


def kernel(x, table):
    tbl = table.T.reshape(2, 8, _RT, 128).transpose(0, 2, 1, 3).reshape(-1)
    xt = x.T.reshape(-1)
    out1 = _embed_gather(xt, tbl)
    out5 = out1.reshape(_NFIELD, 2, 128, 8, 128)
    return out5.transpose(2, 4, 0, 1, 3).reshape(_BATCH, _NFIELD, _DIM)
